# baseline (speedup 1.0000x reference)
import functools

import jax
import jax.numpy as jnp
from jax import lax
from jax.experimental import pallas as pl
from jax.experimental.pallas import tpu as pltpu
from jax.experimental.pallas import tpu_sc as plsc

_C = 96
_PAD = 128
_V = 512 * 512
_B = 1048576
_NC = 2
_NS = 16
_NW = _NC * _NS
_BW = _B // _NW
_CHUNK = 128
_NCH = _BW // _CHUNK
_NBUF = 5
_DEPTH = 4

_BH = 8
_GRID_T = 512 // _BH
_CROWS = _BH * 512 * _C // _PAD


def _transpose_body(img_ref, out_ref):
    x = img_ref[0].reshape(_C, _BH * 512)
    xt = x.T
    xt4 = xt.reshape(_BH * 512 // 4, 4, _C)
    a = jnp.concatenate([xt4[:, 0, :], xt4[:, 1, 0:32]], axis=1)
    b = jnp.concatenate([xt4[:, 1, 32:96], xt4[:, 2, 0:64]], axis=1)
    c = jnp.concatenate([xt4[:, 2, 64:96], xt4[:, 3, :]], axis=1)
    y = jnp.stack([a, b, c], axis=1)
    out_ref[...] = y.reshape(_CROWS, _PAD)


_tc_transpose = pl.pallas_call(
    _transpose_body,
    grid=(_GRID_T,),
    in_specs=[pl.BlockSpec((1, _C, _BH, 512), lambda i: (0, 0, i, 0))],
    out_specs=pl.BlockSpec((_CROWS, _PAD), lambda i: (i, 0)),
    out_shape=jax.ShapeDtypeStruct((_V * _C // _PAD, _PAD), jnp.float32),
)


def _gather_body(table_hbm, idx_hbm, out_hbm, idx_v, rows_v, gsems, osems):
    wid = lax.axis_index("s") * _NC + lax.axis_index("c")
    pltpu.sync_copy(idx_hbm.at[pl.ds(wid * _NCH, _NCH)], idx_v)
    out_base = wid * _BW

    def gather_desc(j, b):
        return pltpu.make_async_copy(
            table_hbm.at[idx_v.at[j]], rows_v.at[b], gsems.at[b])

    def out_desc(j, b):
        return pltpu.make_async_copy(
            rows_v.at[b], out_hbm.at[pl.ds(out_base + j * _CHUNK, _CHUNK)],
            osems.at[b])

    def step(t, carry):
        @pl.when(t < _NCH)
        def _():
            b = t % _NBUF

            @pl.when(t >= _NBUF)
            def _():
                out_desc(t - _NBUF, b).wait()

            gather_desc(t, b).start()

        p = t - _DEPTH

        @pl.when(p >= 0)
        def _():
            bp = p % _NBUF
            gather_desc(p, bp).wait()
            out_desc(p, bp).start()

        return carry

    lax.fori_loop(0, _NCH + _DEPTH, step, 0)

    for b in range(_NBUF):
        j = _NCH - _NBUF + b
        out_desc(j, j % _NBUF).wait()


_sc_gather = functools.partial(
    pl.kernel,
    out_type=jax.ShapeDtypeStruct((_B, _C), jnp.float32),
    compiler_params=pltpu.CompilerParams(use_tc_tiling_on_sc=False),
    mesh=plsc.VectorSubcoreMesh(core_axis_name="c", subcore_axis_name="s"),
    scratch_types=[
        pltpu.VMEM((_NCH, _CHUNK), jnp.int32),
        pltpu.VMEM((_NBUF, _CHUNK, _C), jnp.float32),
        pltpu.SemaphoreType.DMA((_NBUF,)),
        pltpu.SemaphoreType.DMA((_NBUF,)),
    ],
)(_gather_body)


def kernel(img, indices):
    cnt = _tc_transpose(img)
    table = cnt.reshape(_V, _C)
    idx2 = indices.astype(jnp.int32).reshape(_NW * _NCH, _CHUNK)
    return _sc_gather(table, idx2)


# speedup vs baseline: 1.0947x; 1.0947x over previous
import functools

import jax
import jax.numpy as jnp
from jax import lax
from jax.experimental import pallas as pl
from jax.experimental.pallas import tpu as pltpu
from jax.experimental.pallas import tpu_sc as plsc

_C = 96
_PAD = 128
_V = 512 * 512
_B = 1048576
_NC = 2
_NS = 16
_NW = _NC * _NS
_BW = _B // _NW
_PIX = 32
_EX = 3 * _PIX
_NCH = _BW // _PIX
_NBUF = 6
_DEPTH = 4

_BH = 16
_GRID_T = 512 // _BH


def _transpose_body(img_ref, out_ref):
    x = img_ref[0].reshape(_C, _BH * 512)
    out_ref[:, 0:_C] = x.T


_tc_transpose = pl.pallas_call(
    _transpose_body,
    grid=(_GRID_T,),
    in_specs=[pl.BlockSpec((1, _C, _BH, 512), lambda i: (0, 0, i, 0))],
    out_specs=pl.BlockSpec((_BH * 512, _PAD), lambda i: (i, 0)),
    out_shape=jax.ShapeDtypeStruct((_V, _PAD), jnp.float32),
)


def _gather_body(table_hbm, idx_hbm, out_hbm, idx_v, rows_v, gsems, osems):
    wid = lax.axis_index("s") * _NC + lax.axis_index("c")
    pltpu.sync_copy(idx_hbm.at[pl.ds(wid * _NCH, _NCH)], idx_v)
    out_base = 3 * wid * _BW

    def gather_desc(j, b):
        return pltpu.make_async_copy(
            table_hbm.at[idx_v.at[j]], rows_v.at[b], gsems.at[b])

    def out_desc(j, b):
        return pltpu.make_async_copy(
            rows_v.at[b], out_hbm.at[pl.ds(out_base + j * _EX, _EX)],
            osems.at[b])

    def step(t, carry):
        @pl.when(t < _NCH)
        def _():
            b = t % _NBUF

            @pl.when(t >= _NBUF)
            def _():
                out_desc(t - _NBUF, b).wait()

            gather_desc(t, b).start()

        p = t - _DEPTH

        @pl.when(p >= 0)
        def _():
            bp = p % _NBUF
            gather_desc(p, bp).wait()
            out_desc(p, bp).start()

        return carry

    lax.fori_loop(0, _NCH + _DEPTH, step, 0)

    for b in range(_NBUF):
        j = _NCH - _NBUF + b
        out_desc(j, j % _NBUF).wait()


_sc_gather = functools.partial(
    pl.kernel,
    out_type=jax.ShapeDtypeStruct((3 * _B, _PIX), jnp.float32),
    compiler_params=pltpu.CompilerParams(use_tc_tiling_on_sc=False),
    mesh=plsc.VectorSubcoreMesh(core_axis_name="c", subcore_axis_name="s"),
    scratch_types=[
        pltpu.VMEM((_NCH, _EX), jnp.int32),
        pltpu.VMEM((_NBUF, _EX, _PIX), jnp.float32),
        pltpu.SemaphoreType.DMA((_NBUF,)),
        pltpu.SemaphoreType.DMA((_NBUF,)),
    ],
)(_gather_body)


def kernel(img, indices):
    tpad = _tc_transpose(img)
    view32 = tpad.reshape(4 * _V, _PIX)
    idx = indices.astype(jnp.int32).reshape(_NW * _NCH, _PIX)
    idx_ex = (4 * jnp.repeat(idx, 3, axis=1)
              + jnp.tile(jnp.arange(3, dtype=jnp.int32), _PIX)[None, :])
    out32 = _sc_gather(view32, idx_ex)
    return out32.reshape(_B, _C)


# speedup vs baseline: 1.7653x; 1.6126x over previous
import functools

import jax
import jax.numpy as jnp
from jax import lax
from jax.experimental import pallas as pl
from jax.experimental.pallas import tpu as pltpu
from jax.experimental.pallas import tpu_sc as plsc

_C = 96
_PAD = 128
_V = 512 * 512
_B = 1048576
_NC = 2
_NS = 16
_NW = _NC * _NS
_BW = _B // _NW
_CHUNK = 128
_NCH = _BW // _CHUNK
_NBUF = 5
_DEPTH = 4

_BH = 16
_GRID_T = 512 // _BH


def _transpose_body(img_ref, out_ref):
    x = img_ref[0].reshape(_C, _BH * 512)
    out_ref[:, 0:_C] = x.T


_tc_transpose = pl.pallas_call(
    _transpose_body,
    grid=(_GRID_T,),
    in_specs=[pl.BlockSpec((1, _C, _BH, 512), lambda i: (0, 0, i, 0))],
    out_specs=pl.BlockSpec((_BH * 512, _PAD), lambda i: (i, 0)),
    out_shape=jax.ShapeDtypeStruct((_V, _PAD), jnp.float32),
)


def _gather_body(table_hbm, idx_hbm, out_hbm, idx_v, rows_v, gsems, osems):
    wid = lax.axis_index("s") * _NC + lax.axis_index("c")
    pltpu.sync_copy(idx_hbm.at[pl.ds(wid * _NCH, _NCH)], idx_v)
    out_base = wid * _BW

    def gather_desc(j, b):
        return pltpu.make_async_copy(
            table_hbm.at[idx_v.at[j]], rows_v.at[b], gsems.at[b])

    def out_desc(j, b):
        return pltpu.make_async_copy(
            rows_v.at[b], out_hbm.at[pl.ds(out_base + j * _CHUNK, _CHUNK)],
            osems.at[b])

    def step(t, carry):
        @pl.when(t < _NCH)
        def _():
            b = t % _NBUF

            @pl.when(t >= _NBUF)
            def _():
                out_desc(t - _NBUF, b).wait()

            gather_desc(t, b).start()

        p = t - _DEPTH

        @pl.when(p >= 0)
        def _():
            bp = p % _NBUF
            gather_desc(p, bp).wait()
            out_desc(p, bp).start()

        return carry

    lax.fori_loop(0, _NCH + _DEPTH, step, 0)

    for b in range(_NBUF):
        j = _NCH - _NBUF + b
        out_desc(j, j % _NBUF).wait()


_sc_gather = functools.partial(
    pl.kernel,
    out_type=jax.ShapeDtypeStruct((_B, _PAD), jnp.float32),
    mesh=plsc.VectorSubcoreMesh(core_axis_name="c", subcore_axis_name="s"),
    scratch_types=[
        pltpu.VMEM((_NCH, _CHUNK), jnp.int32),
        pltpu.VMEM((_NBUF, _CHUNK, _PAD), jnp.float32),
        pltpu.SemaphoreType.DMA((_NBUF,)),
        pltpu.SemaphoreType.DMA((_NBUF,)),
    ],
)(_gather_body)


def kernel(img, indices):
    table = _tc_transpose(img)
    idx2 = indices.astype(jnp.int32).reshape(_NW * _NCH, _CHUNK)
    return _sc_gather(table, idx2)[:, :_C]
